# baseline (device time: 54187 ns/iter reference)
import jax
import jax.numpy as jnp
from jax import lax
from jax.experimental import pallas as pl
from jax.experimental.pallas import tpu as pltpu

N_DEV = 4


def kernel(x, Wg, Wu, Wd):
    m, k = x.shape
    d = Wd.shape[1]
    mc = m // N_DEV

    def body(x_ref, wg_ref, wu_ref, wd_ref, out_ref,
             st_buf, rs_buf, ag_src, ag_buf,
             rs_send, rs_recv, ag_send, ag_recv):
        my = lax.axis_index("i")

        barrier_sem = pltpu.get_barrier_semaphore()
        for p in range(3):
            pl.semaphore_signal(
                barrier_sem, inc=1,
                device_id=((my + 1 + p) % N_DEV,),
                device_id_type=pl.DeviceIdType.MESH,
            )
        pl.semaphore_wait(barrier_sem, 3)

        def compute_chunk(c):
            rows = pl.ds(c * mc, mc)
            xg = x_ref[rows, :]
            gate = jnp.dot(xg, wg_ref[:, :], preferred_element_type=jnp.float32)
            up = jnp.dot(xg, wu_ref[:, :], preferred_element_type=jnp.float32)
            h = gate * (up * jax.nn.sigmoid(up))
            return jnp.dot(h, wd_ref[:, :], preferred_element_type=jnp.float32)

        rs_rdmas = []
        for p in range(3):
            dst = (my + 1 + p) % N_DEV
            st_buf[p, :, :] = compute_chunk(dst).astype(jnp.bfloat16)
            rdma = pltpu.make_async_remote_copy(
                src_ref=st_buf.at[p],
                dst_ref=rs_buf.at[2 - p],
                send_sem=rs_send.at[p],
                recv_sem=rs_recv.at[2 - p],
                device_id=(dst,),
                device_id_type=pl.DeviceIdType.MESH,
            )
            rdma.start()
            rs_rdmas.append(rdma)

        p_own = compute_chunk(my)

        for q in range(3):
            recv = pltpu.make_async_remote_copy(
                src_ref=st_buf.at[0],
                dst_ref=rs_buf.at[q],
                send_sem=rs_send.at[0],
                recv_sem=rs_recv.at[q],
                device_id=(my,),
                device_id_type=pl.DeviceIdType.MESH,
            )
            recv.wait_recv()

        acc = (
            p_own
            + rs_buf[0, :, :].astype(jnp.float32)
            + rs_buf[1, :, :].astype(jnp.float32)
            + rs_buf[2, :, :].astype(jnp.float32)
        )
        out_ref[pl.ds(my * mc, mc), :] = acc
        ag_src[:, :] = acc.astype(jnp.bfloat16)

        ag_rdmas = []
        for p in range(3):
            dst = (my + 1 + p) % N_DEV
            rdma = pltpu.make_async_remote_copy(
                src_ref=ag_src,
                dst_ref=ag_buf.at[2 - p],
                send_sem=ag_send.at[p],
                recv_sem=ag_recv.at[2 - p],
                device_id=(dst,),
                device_id_type=pl.DeviceIdType.MESH,
            )
            rdma.start()
            ag_rdmas.append(rdma)

        for q in range(3):
            recv = pltpu.make_async_remote_copy(
                src_ref=ag_src,
                dst_ref=ag_buf.at[q],
                send_sem=ag_send.at[0],
                recv_sem=ag_recv.at[q],
                device_id=(my,),
                device_id_type=pl.DeviceIdType.MESH,
            )
            recv.wait_recv()
            src_chunk = (my + 1 + q) % N_DEV
            out_ref[pl.ds(src_chunk * mc, mc), :] = ag_buf[q, :, :].astype(
                jnp.float32
            )

        for rdma in rs_rdmas + ag_rdmas:
            rdma.wait_send()

    return pl.pallas_call(
        body,
        out_shape=jax.ShapeDtypeStruct((m, d), jnp.float32),
        in_specs=[pl.BlockSpec(memory_space=pltpu.VMEM)] * 4,
        out_specs=pl.BlockSpec(memory_space=pltpu.VMEM),
        scratch_shapes=[
            pltpu.VMEM((3, mc, d), jnp.bfloat16),
            pltpu.VMEM((3, mc, d), jnp.bfloat16),
            pltpu.VMEM((mc, d), jnp.bfloat16),
            pltpu.VMEM((3, mc, d), jnp.bfloat16),
            pltpu.SemaphoreType.DMA((3,)),
            pltpu.SemaphoreType.DMA((3,)),
            pltpu.SemaphoreType.DMA((3,)),
            pltpu.SemaphoreType.DMA((3,)),
        ],
        compiler_params=pltpu.CompilerParams(
            collective_id=0, vmem_limit_bytes=100 * 1024 * 1024
        ),
    )(x, Wg, Wu, Wd)
